# baseline (speedup 1.0000x reference)
import functools

import jax
import jax.numpy as jnp
from jax import lax
from jax.experimental import pallas as pl
from jax.experimental.pallas import tpu as pltpu
from jax.experimental.pallas import tpu_sc as plsc

NTOK, DIM, INTER, NEXP = 4096, 1024, 4096, 8
BLK = 512
KBLK = 1024
K = INTER // KBLK
CAP = NEXP * (NTOK + BLK)
NBMAX = CAP // BLK

NC, NS = 2, 16
NW = NC * NS
GCHUNK = 64
ROWS_PER_TILE = CAP // NW
TG = 8


def _sc_mesh():
    return plsc.VectorSubcoreMesh(
        core_axis_name="c", subcore_axis_name="s",
        num_cores=NC, num_subcores=NS)


def _routing(expert_indices, expert_weights):
    m = expert_indices != 0
    mi = m.astype(jnp.int32)
    cnt = jnp.sum(mi, axis=0)
    nblk = cnt // BLK + 1
    padded = nblk * BLK
    off = jnp.concatenate([jnp.zeros((1,), jnp.int32), jnp.cumsum(padded)])
    rank = jnp.cumsum(mi, axis=0) - mi
    dest = off[:NEXP][None, :] + rank
    safe_dest = jnp.where(m, dest, CAP)
    tok_ids = jnp.broadcast_to(
        jnp.arange(NTOK, dtype=jnp.int32)[:, None], (NTOK, NEXP))
    perm_tok = jnp.zeros((CAP + 1,), jnp.int32).at[
        safe_dest.reshape(-1)].set(tok_ids.reshape(-1))[:CAP]
    row_w = jnp.zeros((CAP + 1,), jnp.float32).at[
        safe_dest.reshape(-1)].set(expert_weights.reshape(-1))[:CAP]
    inv_idx = jnp.where(m, dest, (off[:NEXP] + cnt)[None, :]).astype(jnp.int32)
    blk_cum = jnp.cumsum(nblk)
    bids = jnp.arange(NBMAX, dtype=jnp.int32)
    be = jnp.searchsorted(blk_cum, bids, side="right").astype(jnp.int32)
    bv = (be < NEXP).astype(jnp.int32)
    be = jnp.minimum(be, NEXP - 1)
    bx = jnp.where(bv == 1, bids, 0)
    by = jnp.where(bv == 1, bids, NBMAX - 1)
    sp = jnp.stack([be, bv, bx, by])
    return perm_tok, row_w, inv_idx.reshape(-1), sp


def _sc_gather(x, perm_tok):
    @functools.partial(
        pl.kernel,
        out_type=jax.ShapeDtypeStruct((CAP, DIM), jnp.float32),
        mesh=_sc_mesh(),
        scratch_types=[
            pltpu.VMEM((GCHUNK,), jnp.int32),
            pltpu.VMEM((GCHUNK, DIM), jnp.float32),
            pltpu.SemaphoreType.DMA,
        ],
    )
    def gather_k(x_hbm, idx_hbm, xp_hbm, idx_v, rows_v, sem):
        wid = lax.axis_index("s") * NC + lax.axis_index("c")
        base = wid * ROWS_PER_TILE

        def body(i, carry):
            o = base + i * GCHUNK
            pltpu.sync_copy(idx_hbm.at[pl.ds(o, GCHUNK)], idx_v)
            pltpu.async_copy(x_hbm.at[idx_v], rows_v, sem).wait()
            pltpu.sync_copy(rows_v, xp_hbm.at[pl.ds(o, GCHUNK)])
            return carry

        lax.fori_loop(0, ROWS_PER_TILE // GCHUNK, body, 0)

    return gather_k(x, perm_tok)


def _ffn_body(sp_ref, x_ref, w1_ref, w3_ref, w2_ref, rw_ref, y_ref, acc_ref):
    b = pl.program_id(0)
    k = pl.program_id(1)

    @pl.when(k == 0)
    def _():
        acc_ref[...] = jnp.zeros_like(acc_ref)

    @pl.when(sp_ref[1, b] != 0)
    def _():
        x = x_ref[...].astype(jnp.bfloat16)
        dn = (((1,), (1,)), ((), ()))
        x1 = lax.dot_general(x, w1_ref[0], dn,
                             preferred_element_type=jnp.float32)
        x3 = lax.dot_general(x, w3_ref[0], dn,
                             preferred_element_type=jnp.float32)
        h = x1 * lax.logistic(x1) * x3
        acc_ref[...] += lax.dot_general(h.astype(jnp.bfloat16), w2_ref[0], dn,
                                        preferred_element_type=jnp.float32)

    @pl.when(k == K - 1)
    def _():
        y_ref[...] = acc_ref[...] * rw_ref[0, 0, :][:, None]


def _grouped_ffn(x_perm, w1_bf, w3_bf, w2_bf, row_w3, sp):
    grid_spec = pltpu.PrefetchScalarGridSpec(
        num_scalar_prefetch=1,
        grid=(NBMAX, K),
        in_specs=[
            pl.BlockSpec((BLK, DIM), lambda b, k, sp: (sp[2, b], 0)),
            pl.BlockSpec((1, KBLK, DIM), lambda b, k, sp: (sp[0, b], k, 0)),
            pl.BlockSpec((1, KBLK, DIM), lambda b, k, sp: (sp[0, b], k, 0)),
            pl.BlockSpec((1, DIM, KBLK), lambda b, k, sp: (sp[0, b], 0, k)),
            pl.BlockSpec((1, 1, BLK), lambda b, k, sp: (sp[2, b], 0, 0)),
        ],
        out_specs=pl.BlockSpec((BLK, DIM), lambda b, k, sp: (sp[3, b], 0)),
        scratch_shapes=[pltpu.VMEM((BLK, DIM), jnp.float32)],
    )
    return pl.pallas_call(
        _ffn_body,
        grid_spec=grid_spec,
        out_shape=jax.ShapeDtypeStruct((CAP, DIM), jnp.float32),
        compiler_params=pltpu.CompilerParams(
            dimension_semantics=("arbitrary", "arbitrary")),
    )(sp, x_perm, w1_bf, w3_bf, w2_bf, row_w3)


def _sc_combine(y, inv_idx):
    toks_per_tile = NTOK // NW
    idxc = TG * NEXP

    @functools.partial(
        pl.kernel,
        out_type=jax.ShapeDtypeStruct((NTOK, DIM), jnp.float32),
        mesh=_sc_mesh(),
        scratch_types=[
            pltpu.VMEM((idxc,), jnp.int32),
            pltpu.VMEM((idxc, DIM), jnp.float32),
            pltpu.VMEM((TG, DIM), jnp.float32),
            pltpu.SemaphoreType.DMA,
        ],
    )
    def combine_k(y_hbm, inv_hbm, out_hbm, idx_v, rows_v, out_v, sem):
        wid = lax.axis_index("s") * NC + lax.axis_index("c")
        tbase = wid * toks_per_tile

        def grp(g, carry):
            t0 = tbase + g * TG
            pltpu.sync_copy(inv_hbm.at[pl.ds(t0 * NEXP, idxc)], idx_v)
            pltpu.async_copy(y_hbm.at[idx_v], rows_v, sem).wait()

            def chunk(i, c2):
                t = i // (DIM // 16)
                c = (i % (DIM // 16)) * 16
                acc = rows_v[t * NEXP, pl.ds(c, 16)]
                for e in range(1, NEXP):
                    acc = acc + rows_v[t * NEXP + e, pl.ds(c, 16)]
                out_v[t, pl.ds(c, 16)] = acc
                return c2

            lax.fori_loop(0, TG * (DIM // 16), chunk, 0)
            pltpu.sync_copy(out_v, out_hbm.at[pl.ds(t0, TG)])
            return carry

        lax.fori_loop(0, toks_per_tile // TG, grp, 0)

    return combine_k(y, inv_idx)


def kernel(x, expert_indices, expert_weights, w1, w2, w3):
    perm_tok, row_w, inv_idx, sp = _routing(expert_indices, expert_weights)
    w1_bf = w1.astype(jnp.bfloat16)
    w3_bf = w3.astype(jnp.bfloat16)
    w2_bf = w2.astype(jnp.bfloat16)
    row_w3 = row_w.reshape(NBMAX, 1, BLK)
    x_perm = _sc_gather(x, perm_tok)
    y = _grouped_ffn(x_perm, w1_bf, w3_bf, w2_bf, row_w3, sp)
    return _sc_combine(y, inv_idx)
